# baseline (speedup 1.0000x reference)
import jax
import jax.numpy as jnp
from jax import lax
from jax.experimental import pallas as pl
from jax.experimental.pallas import tpu as pltpu, tpu_sc as plsc

import functools


_LANES = 16


def _make_sc_gather(n_layers, B, S, D):
    assert 8 + B <= _LANES and D % 128 == 0
    base = (n_layers - 1) * B * S

    mesh = plsc.VectorSubcoreMesh(core_axis_name="c", subcore_axis_name="s",
                                  num_cores=1, num_subcores=3)

    @functools.partial(
        pl.kernel,
        mesh=mesh,
        out_type=jax.ShapeDtypeStruct((B, 3 * D), jnp.float32),
        scratch_types=[
            pltpu.VMEM((_LANES,), jnp.int32),
            pltpu.VMEM((_LANES,), jnp.int32),
            pltpu.VMEM((B, D), jnp.float32),
            pltpu.SemaphoreType.DMA,
        ],
    )
    def sc_gather(table_hbm, idx1_hbm, idx2_hbm, out_hbm,
                  pos_v, ridx, rows, sem):
        tid = lax.axis_index("s")
        ramp = base + (lax.iota(jnp.int32, _LANES) - 8) * S

        def column(col, idx_hbm):
            if idx_hbm is not None:
                pltpu.sync_copy(idx_hbm, pos_v.at[pl.ds(8, B)])
                ridx[...] = ramp + pos_v[...]
            else:
                ridx[...] = ramp
            pltpu.async_copy(
                table_hbm.at[ridx.at[pl.ds(8, B)]], rows, sem).wait()
            pltpu.sync_copy(rows, out_hbm.at[:, pl.ds(col * D, D)])

        @pl.when(tid == 0)
        def _():
            column(0, None)

        @pl.when(tid == 1)
        def _():
            column(1, idx1_hbm)

        @pl.when(tid == 2)
        def _():
            column(2, idx2_hbm)

    return sc_gather


def kernel(feature, idx1, idx2):
    n_layers, B, S, D = feature.shape
    table = feature.reshape(n_layers * B * S, D)
    sc_gather = _make_sc_gather(n_layers, B, S, D)
    return sc_gather(table, idx1.astype(jnp.int32), idx2.astype(jnp.int32))
